# baseline (speedup 1.0000x reference)
import functools

import jax
import jax.numpy as jnp
from jax import lax
from jax.experimental import pallas as pl
from jax.experimental.pallas import tpu as pltpu
from jax.experimental.pallas import tpu_sc as plsc

N_NODES = 10000
N_EDGES = 320000
CH = 128

NC = 2
NS = 16
NW = NC * NS
CHUNK = 128
NCHUNKS = 80
HALF = NCHUNKS // 2
E_PAD = NW * NCHUNKS * CHUNK
N_PAD = 10240
ROWS_PER_TILE = N_PAD // NS
ZROWS = 128
LANES = 16


_mesh = plsc.VectorSubcoreMesh(core_axis_name="c", subcore_axis_name="s")


@functools.partial(
    pl.kernel,
    out_type=jax.ShapeDtypeStruct((NC, N_PAD, CH), jnp.float32),
    mesh=_mesh,
    scratch_types=[
        pltpu.VMEM((HALF, CHUNK), jnp.int32),
        pltpu.VMEM((HALF, CHUNK), jnp.int32),
        pltpu.VMEM((2, CHUNK, CH), jnp.float32),
        pltpu.VMEM_SHARED((N_PAD, CH), jnp.float32),
        pltpu.SemaphoreType.DMA,
        pltpu.SemaphoreType.DMA,
    ],
)
def _sc_segment_sum(row_hbm, col_hbm, x_hbm, out_hbm,
                    rowv, colv, rows, acc, isem, gsem):
    c = lax.axis_index("c")
    s = lax.axis_index("s")
    wid = c * NS + s

    def _zero_row(i, carry):
        zero = jnp.zeros((LANES,), jnp.float32)
        for j in range(CH // LANES):
            rows[0, i, pl.ds(j * LANES, LANES)] = zero
        return carry
    lax.fori_loop(0, CHUNK, _zero_row, 0)
    for k in range(ROWS_PER_TILE // CHUNK):
        pltpu.sync_copy(rows.at[0],
                        acc.at[pl.ds(s * ROWS_PER_TILE + k * CHUNK, CHUNK)])
    plsc.subcore_barrier()

    for h in range(2):
        base = wid * NCHUNKS + h * HALF
        pltpu.async_copy(row_hbm.at[pl.ds(base, HALF)], rowv, isem)
        pltpu.async_copy(col_hbm.at[pl.ds(base, HALF)], colv, isem)
        pltpu.make_async_copy(row_hbm.at[pl.ds(base, HALF)], rowv, isem).wait()
        pltpu.make_async_copy(col_hbm.at[pl.ds(base, HALF)], colv, isem).wait()

        pltpu.async_copy(x_hbm.at[colv.at[0]], rows.at[0], gsem)

        def _chunk(i, carry):
            buf = lax.rem(i, 2)
            nbuf = lax.rem(i + 1, 2)
            nxt = lax.rem(i + 1, HALF)
            pltpu.make_async_copy(x_hbm.at[colv.at[i]], rows.at[buf],
                                  gsem).wait()
            pltpu.async_copy(x_hbm.at[colv.at[nxt]], rows.at[nbuf], gsem)
            pltpu.sync_copy(rows.at[buf], acc.at[rowv.at[i]], add=True)
            return carry
        lax.fori_loop(0, HALF, _chunk, 0)
        pltpu.make_async_copy(x_hbm.at[colv.at[0]], rows.at[lax.rem(HALF, 2)],
                              gsem).wait()

    plsc.subcore_barrier()
    pltpu.sync_copy(acc.at[pl.ds(s * ROWS_PER_TILE, ROWS_PER_TILE)],
                    out_hbm.at[c, pl.ds(s * ROWS_PER_TILE, ROWS_PER_TILE)])


def _tc_matmul_body(p_ref, w_ref, o_ref):
    seg = p_ref[0, :N_NODES, :] + p_ref[1, :N_NODES, :]
    o_ref[...] = lax.dot_general(
        seg, w_ref[...], (((1,), (1,)), ((), ())),
        preferred_element_type=jnp.float32,
        precision=lax.Precision.HIGHEST)


def kernel(x, edge_index, edge_attr, W_fc, W_edge, W_att):
    npad = E_PAD - N_EDGES
    pad_rows = N_NODES + jnp.arange(npad, dtype=jnp.int32) % (N_PAD - N_NODES)
    row = jnp.concatenate(
        [edge_index[0].astype(jnp.int32),
         pad_rows]).reshape(NW * NCHUNKS, CHUNK)
    col = jnp.concatenate(
        [edge_index[1].astype(jnp.int32),
         jnp.zeros((npad,), jnp.int32)]).reshape(NW * NCHUNKS, CHUNK)
    partials = _sc_segment_sum(row, col, x)
    z = pl.pallas_call(
        _tc_matmul_body,
        out_shape=jax.ShapeDtypeStruct((N_NODES, CH), jnp.float32),
    )(partials, W_fc)
    return z


# speedup vs baseline: 1.0012x; 1.0012x over previous
import functools

import jax
import jax.numpy as jnp
from jax import lax
from jax.experimental import pallas as pl
from jax.experimental.pallas import tpu as pltpu
from jax.experimental.pallas import tpu_sc as plsc

N_NODES = 10000
N_EDGES = 320000
CH = 128

NC = 2
NS = 16
NW = NC * NS
CHUNK = 128
NCHUNKS = 80
HALF = NCHUNKS // 2
E_PAD = NW * NCHUNKS * CHUNK
N_PAD = 10240
ROWS_PER_TILE = N_PAD // NS
ZROWS = 128
LANES = 16


_mesh = plsc.VectorSubcoreMesh(core_axis_name="c", subcore_axis_name="s")


@functools.partial(
    pl.kernel,
    out_type=jax.ShapeDtypeStruct((NC, N_PAD, CH), jnp.float32),
    mesh=_mesh,
    scratch_types=[
        pltpu.VMEM((HALF, CHUNK), jnp.int32),
        pltpu.VMEM((HALF, CHUNK), jnp.int32),
        pltpu.VMEM((2, CHUNK, CH), jnp.float32),
        pltpu.VMEM_SHARED((N_PAD, CH), jnp.float32),
        pltpu.SemaphoreType.DMA,
        pltpu.SemaphoreType.DMA,
    ],
)
def _sc_segment_sum(row_hbm, col_hbm, x_hbm, out_hbm,
                    rowv, colv, rows, acc, isem, gsem):
    c = lax.axis_index("c")
    s = lax.axis_index("s")
    wid = s * NC + c

    def _zero_row(i, carry):
        zero = jnp.zeros((LANES,), jnp.float32)
        for j in range(CH // LANES):
            rows[0, i, pl.ds(j * LANES, LANES)] = zero
        return carry
    lax.fori_loop(0, CHUNK, _zero_row, 0)
    for k in range(ROWS_PER_TILE // CHUNK):
        pltpu.sync_copy(rows.at[0],
                        acc.at[pl.ds(s * ROWS_PER_TILE + k * CHUNK, CHUNK)])
    plsc.subcore_barrier()

    for h in range(2):
        base = wid * NCHUNKS + h * HALF
        pltpu.async_copy(row_hbm.at[pl.ds(base, HALF)], rowv, isem)
        pltpu.async_copy(col_hbm.at[pl.ds(base, HALF)], colv, isem)
        pltpu.make_async_copy(row_hbm.at[pl.ds(base, HALF)], rowv, isem).wait()
        pltpu.make_async_copy(col_hbm.at[pl.ds(base, HALF)], colv, isem).wait()

        pltpu.async_copy(x_hbm.at[colv.at[0]], rows.at[0], gsem)

        def _chunk(i, carry):
            buf = lax.rem(i, 2)
            nbuf = lax.rem(i + 1, 2)
            nxt = lax.rem(i + 1, HALF)
            pltpu.make_async_copy(x_hbm.at[colv.at[i]], rows.at[buf],
                                  gsem).wait()
            pltpu.async_copy(x_hbm.at[colv.at[nxt]], rows.at[nbuf], gsem)
            pltpu.sync_copy(rows.at[buf], acc.at[rowv.at[i]], add=True)
            return carry
        lax.fori_loop(0, HALF, _chunk, 0)
        pltpu.make_async_copy(x_hbm.at[colv.at[0]], rows.at[lax.rem(HALF, 2)],
                              gsem).wait()

    plsc.subcore_barrier()
    pltpu.sync_copy(acc.at[pl.ds(s * ROWS_PER_TILE, ROWS_PER_TILE)],
                    out_hbm.at[c, pl.ds(s * ROWS_PER_TILE, ROWS_PER_TILE)])


def _tc_matmul_body(p_ref, w_ref, o_ref):
    seg = p_ref[0, :N_NODES, :] + p_ref[1, :N_NODES, :]
    o_ref[...] = lax.dot_general(
        seg, w_ref[...], (((1,), (1,)), ((), ())),
        preferred_element_type=jnp.float32,
        precision=lax.Precision.HIGHEST)


def kernel(x, edge_index, edge_attr, W_fc, W_edge, W_att):
    npad = E_PAD - N_EDGES
    pad_rows = N_NODES + jnp.arange(npad, dtype=jnp.int32) % (N_PAD - N_NODES)
    row = jnp.concatenate(
        [edge_index[0].astype(jnp.int32),
         pad_rows]).reshape(NW * NCHUNKS, CHUNK)
    col = jnp.concatenate(
        [edge_index[1].astype(jnp.int32),
         jnp.zeros((npad,), jnp.int32)]).reshape(NW * NCHUNKS, CHUNK)
    partials = _sc_segment_sum(row, col, x)
    z = pl.pallas_call(
        _tc_matmul_body,
        out_shape=jax.ShapeDtypeStruct((N_NODES, CH), jnp.float32),
    )(partials, W_fc)
    return z


# speedup vs baseline: 1.0949x; 1.0936x over previous
import functools

import jax
import jax.numpy as jnp
from jax import lax
from jax.experimental import pallas as pl
from jax.experimental.pallas import tpu as pltpu
from jax.experimental.pallas import tpu_sc as plsc

N_NODES = 10000
N_EDGES = 320000
CH = 128

NC = 2
NS = 16
NW = NC * NS
CHUNK = 128
TOTAL_CHUNKS = 2560
N0 = 128
N1 = (TOTAL_CHUNKS - NS * N0) // NS
IB = 16
E_PAD = TOTAL_CHUNKS * CHUNK
N_PAD = 10240
ROWS_PER_TILE = N_PAD // NS
LANES = 16


_mesh = plsc.VectorSubcoreMesh(core_axis_name="c", subcore_axis_name="s")


@functools.partial(
    pl.kernel,
    out_type=jax.ShapeDtypeStruct((NC, N_PAD, CH), jnp.float32),
    mesh=_mesh,
    scratch_types=[
        pltpu.VMEM((2, IB, CHUNK), jnp.int32),
        pltpu.VMEM((2, IB, CHUNK), jnp.int32),
        pltpu.VMEM((2, CHUNK, CH), jnp.float32),
        pltpu.VMEM_SHARED((N_PAD, CH), jnp.float32),
        pltpu.SemaphoreType.DMA,
        pltpu.SemaphoreType.DMA,
    ],
)
def _sc_segment_sum(row_hbm, col_hbm, x_hbm, out_hbm,
                    rowv, colv, rows, acc, isem, gsem):
    c = lax.axis_index("c")
    s = lax.axis_index("s")
    cbase = jnp.where(c == 0, s * N0, NS * N0 + s * N1)
    nblocks = jnp.where(c == 0, N0 // IB, N1 // IB)

    def _zero_row(i, carry):
        zero = jnp.zeros((LANES,), jnp.float32)
        for j in range(CH // LANES):
            rows[0, i, pl.ds(j * LANES, LANES)] = zero
        return carry
    lax.fori_loop(0, CHUNK, _zero_row, 0)
    for k in range(ROWS_PER_TILE // CHUNK):
        pltpu.sync_copy(rows.at[0],
                        acc.at[pl.ds(s * ROWS_PER_TILE + k * CHUNK, CHUNK)])
    plsc.subcore_barrier()

    pltpu.async_copy(row_hbm.at[pl.ds(cbase, IB)], rowv.at[0], isem)
    pltpu.async_copy(col_hbm.at[pl.ds(cbase, IB)], colv.at[0], isem)

    def _block(b, carry):
        bbuf = lax.rem(b, 2)
        nbbuf = lax.rem(b + 1, 2)
        nb = jnp.minimum(b + 1, nblocks - 1)
        pltpu.make_async_copy(row_hbm.at[pl.ds(cbase, IB)], rowv.at[bbuf],
                              isem).wait()
        pltpu.make_async_copy(col_hbm.at[pl.ds(cbase, IB)], colv.at[bbuf],
                              isem).wait()
        pltpu.async_copy(row_hbm.at[pl.ds(cbase + nb * IB, IB)],
                         rowv.at[nbbuf], isem)
        pltpu.async_copy(col_hbm.at[pl.ds(cbase + nb * IB, IB)],
                         colv.at[nbbuf], isem)

        pltpu.async_copy(x_hbm.at[colv.at[bbuf].at[0]], rows.at[0], gsem)

        def _chunk(i, carry2):
            buf = lax.rem(i, 2)
            nbuf = lax.rem(i + 1, 2)
            nxt = jnp.minimum(i + 1, IB - 1)
            pltpu.make_async_copy(x_hbm.at[colv.at[bbuf].at[i]], rows.at[buf],
                                  gsem).wait()
            pltpu.async_copy(x_hbm.at[colv.at[bbuf].at[nxt]], rows.at[nbuf],
                             gsem)
            pltpu.sync_copy(rows.at[buf], acc.at[rowv.at[bbuf].at[i]],
                            add=True)
            return carry2
        lax.fori_loop(0, IB, _chunk, 0)
        pltpu.make_async_copy(x_hbm.at[colv.at[bbuf].at[0]],
                              rows.at[lax.rem(IB, 2)], gsem).wait()
        return carry
    lax.fori_loop(0, nblocks, _block, 0)
    pltpu.make_async_copy(row_hbm.at[pl.ds(cbase, IB)], rowv.at[0], isem).wait()
    pltpu.make_async_copy(col_hbm.at[pl.ds(cbase, IB)], colv.at[0], isem).wait()

    plsc.subcore_barrier()
    pltpu.sync_copy(acc.at[pl.ds(s * ROWS_PER_TILE, ROWS_PER_TILE)],
                    out_hbm.at[c, pl.ds(s * ROWS_PER_TILE, ROWS_PER_TILE)])


def _tc_matmul_body(p_ref, w_ref, o_ref):
    seg = p_ref[0, :N_NODES, :] + p_ref[1, :N_NODES, :]
    o_ref[...] = lax.dot_general(
        seg, w_ref[...], (((1,), (1,)), ((), ())),
        preferred_element_type=jnp.float32,
        precision=lax.Precision.HIGHEST)


def kernel(x, edge_index, edge_attr, W_fc, W_edge, W_att):
    npad = E_PAD - N_EDGES
    pad_rows = N_NODES + jnp.arange(npad, dtype=jnp.int32) % (N_PAD - N_NODES)
    row = jnp.concatenate(
        [edge_index[0].astype(jnp.int32),
         pad_rows]).reshape(TOTAL_CHUNKS, CHUNK)
    col = jnp.concatenate(
        [edge_index[1].astype(jnp.int32),
         jnp.zeros((npad,), jnp.int32)]).reshape(TOTAL_CHUNKS, CHUNK)
    partials = _sc_segment_sum(row, col, x)
    z = pl.pallas_call(
        _tc_matmul_body,
        out_shape=jax.ShapeDtypeStruct((N_NODES, CH), jnp.float32),
    )(partials, W_fc)
    return z


# speedup vs baseline: 1.0952x; 1.0002x over previous
import functools

import jax
import jax.numpy as jnp
from jax import lax
from jax.experimental import pallas as pl
from jax.experimental.pallas import tpu as pltpu
from jax.experimental.pallas import tpu_sc as plsc

N_NODES = 10000
N_EDGES = 320000
CH = 128

NC = 2
NS = 16
NW = NC * NS
CHUNK = 128
TOTAL_CHUNKS = 2560
N0 = 128
N1 = (TOTAL_CHUNKS - NS * N0) // NS
IB = 16
E_PAD = TOTAL_CHUNKS * CHUNK
N_PAD = 10240
ROWS_PER_TILE = N_PAD // NS
LANES = 16


_mesh = plsc.VectorSubcoreMesh(core_axis_name="c", subcore_axis_name="s")


@functools.partial(
    pl.kernel,
    out_type=jax.ShapeDtypeStruct((NC, N_PAD, CH), jnp.float32),
    mesh=_mesh,
    scratch_types=[
        pltpu.VMEM((2, IB, CHUNK), jnp.int32),
        pltpu.VMEM((2, IB, CHUNK), jnp.int32),
        pltpu.VMEM((2, CHUNK, CH), jnp.float32),
        pltpu.VMEM_SHARED((N_PAD, CH), jnp.float32),
        pltpu.SemaphoreType.DMA,
        pltpu.SemaphoreType.DMA,
    ],
)
def _sc_segment_sum(row_hbm, col_hbm, x_hbm, out_hbm,
                    rowv, colv, rows, acc, isem, gsem):
    c = lax.axis_index("c")
    s = lax.axis_index("s")
    cbase = jnp.where(c == 0, s * N0, NS * N0 + s * N1)
    nblocks = jnp.where(c == 0, N0 // IB, N1 // IB)

    with jax.named_scope("zero_init"):
        def _zero_row(i, carry):
            zero = jnp.zeros((LANES,), jnp.float32)
            for j in range(CH // LANES):
                rows[0, i, pl.ds(j * LANES, LANES)] = zero
            return carry
        lax.fori_loop(0, CHUNK, _zero_row, 0)
        for k in range(ROWS_PER_TILE // CHUNK):
            pltpu.sync_copy(rows.at[0],
                            acc.at[pl.ds(s * ROWS_PER_TILE + k * CHUNK, CHUNK)])
        plsc.subcore_barrier()

    edge_scope = jax.named_scope("edge_loop")
    edge_scope.__enter__()
    pltpu.async_copy(row_hbm.at[pl.ds(cbase, IB)], rowv.at[0], isem)
    pltpu.async_copy(col_hbm.at[pl.ds(cbase, IB)], colv.at[0], isem)

    def _block(b, carry):
        bbuf = lax.rem(b, 2)
        nbbuf = lax.rem(b + 1, 2)
        nb = jnp.minimum(b + 1, nblocks - 1)
        pltpu.make_async_copy(row_hbm.at[pl.ds(cbase, IB)], rowv.at[bbuf],
                              isem).wait()
        pltpu.make_async_copy(col_hbm.at[pl.ds(cbase, IB)], colv.at[bbuf],
                              isem).wait()
        pltpu.async_copy(row_hbm.at[pl.ds(cbase + nb * IB, IB)],
                         rowv.at[nbbuf], isem)
        pltpu.async_copy(col_hbm.at[pl.ds(cbase + nb * IB, IB)],
                         colv.at[nbbuf], isem)

        pltpu.async_copy(x_hbm.at[colv.at[bbuf].at[0]], rows.at[0], gsem)

        def _chunk(i, carry2):
            buf = lax.rem(i, 2)
            nbuf = lax.rem(i + 1, 2)
            nxt = jnp.minimum(i + 1, IB - 1)
            pltpu.make_async_copy(x_hbm.at[colv.at[bbuf].at[i]], rows.at[buf],
                                  gsem).wait()
            pltpu.async_copy(x_hbm.at[colv.at[bbuf].at[nxt]], rows.at[nbuf],
                             gsem)
            pltpu.sync_copy(rows.at[buf], acc.at[rowv.at[bbuf].at[i]],
                            add=True)
            return carry2
        lax.fori_loop(0, IB, _chunk, 0)
        pltpu.make_async_copy(x_hbm.at[colv.at[bbuf].at[0]],
                              rows.at[lax.rem(IB, 2)], gsem).wait()
        return carry
    lax.fori_loop(0, nblocks, _block, 0)
    pltpu.make_async_copy(row_hbm.at[pl.ds(cbase, IB)], rowv.at[0], isem).wait()
    pltpu.make_async_copy(col_hbm.at[pl.ds(cbase, IB)], colv.at[0], isem).wait()
    edge_scope.__exit__(None, None, None)

    with jax.named_scope("writeback"):
        plsc.subcore_barrier()
        pltpu.sync_copy(acc.at[pl.ds(s * ROWS_PER_TILE, ROWS_PER_TILE)],
                        out_hbm.at[c, pl.ds(s * ROWS_PER_TILE, ROWS_PER_TILE)])


def _tc_matmul_body(p_ref, w_ref, o_ref):
    seg = p_ref[0, :N_NODES, :] + p_ref[1, :N_NODES, :]
    o_ref[...] = lax.dot_general(
        seg, w_ref[...], (((1,), (1,)), ((), ())),
        preferred_element_type=jnp.float32,
        precision=lax.Precision.HIGHEST)


def kernel(x, edge_index, edge_attr, W_fc, W_edge, W_att):
    npad = E_PAD - N_EDGES
    pad_rows = N_NODES + jnp.arange(npad, dtype=jnp.int32) % (N_PAD - N_NODES)
    row = jnp.concatenate(
        [edge_index[0].astype(jnp.int32),
         pad_rows]).reshape(TOTAL_CHUNKS, CHUNK)
    col = jnp.concatenate(
        [edge_index[1].astype(jnp.int32),
         jnp.zeros((npad,), jnp.int32)]).reshape(TOTAL_CHUNKS, CHUNK)
    partials = _sc_segment_sum(row, col, x)
    z = pl.pallas_call(
        _tc_matmul_body,
        out_shape=jax.ShapeDtypeStruct((N_NODES, CH), jnp.float32),
    )(partials, W_fc)
    return z


# speedup vs baseline: 1.0969x; 1.0016x over previous
import functools

import jax
import jax.numpy as jnp
from jax import lax
from jax.experimental import pallas as pl
from jax.experimental.pallas import tpu as pltpu
from jax.experimental.pallas import tpu_sc as plsc

N_NODES = 10000
N_EDGES = 320000
CH = 128

NC = 2
NS = 16
NW = NC * NS
CHUNK = 128
TOTAL_CHUNKS = 2560
N0 = 128
N1 = (TOTAL_CHUNKS - NS * N0) // NS
IB = 16
E_PAD = TOTAL_CHUNKS * CHUNK
N_PAD = 10240
ROWS_PER_TILE = N_PAD // NS
LANES = 16


_mesh = plsc.VectorSubcoreMesh(core_axis_name="c", subcore_axis_name="s")


@functools.partial(
    pl.kernel,
    out_type=jax.ShapeDtypeStruct((NC * N_PAD, CH), jnp.float32),
    mesh=_mesh,
    scratch_types=[
        pltpu.VMEM((2, IB, CHUNK), jnp.int32),
        pltpu.VMEM((2, IB, CHUNK), jnp.int32),
        pltpu.VMEM((2, CHUNK, CH), jnp.float32),
        pltpu.VMEM((2, CHUNK), jnp.int32),
        pltpu.VMEM_SHARED((N_PAD, CH), jnp.float32),
        pltpu.SemaphoreType.DMA,
        pltpu.SemaphoreType.DMA,
        pltpu.SemaphoreType.DMA,
    ],
)
def _sc_segment_sum(row_hbm, col_hbm, x_hbm, out_hbm,
                    rowv, colv, rows, widx, acc, isem, gsem, ssem):
    c = lax.axis_index("c")
    s = lax.axis_index("s")
    cbase = jnp.where(c == 0, s * N0, NS * N0 + s * N1)
    nblocks = jnp.where(c == 0, N0 // IB, N1 // IB)

    with jax.named_scope("zero_init"):
        def _zero_row(i, carry):
            zero = jnp.zeros((LANES,), jnp.float32)
            for j in range(CH // LANES):
                rows[0, i, pl.ds(j * LANES, LANES)] = zero
            return carry
        lax.fori_loop(0, CHUNK, _zero_row, 0)
        for k in range(ROWS_PER_TILE // CHUNK):
            pltpu.sync_copy(rows.at[0],
                            acc.at[pl.ds(s * ROWS_PER_TILE + k * CHUNK, CHUNK)])
        plsc.subcore_barrier()

    edge_scope = jax.named_scope("edge_loop")
    edge_scope.__enter__()
    pltpu.async_copy(row_hbm.at[pl.ds(cbase, IB)], rowv.at[0], isem)
    pltpu.async_copy(col_hbm.at[pl.ds(cbase, IB)], colv.at[0], isem)

    def _block(b, carry):
        bbuf = lax.rem(b, 2)
        nbbuf = lax.rem(b + 1, 2)
        nb = jnp.minimum(b + 1, nblocks - 1)
        pltpu.make_async_copy(row_hbm.at[pl.ds(cbase, IB)], rowv.at[bbuf],
                              isem).wait()
        pltpu.make_async_copy(col_hbm.at[pl.ds(cbase, IB)], colv.at[bbuf],
                              isem).wait()
        pltpu.async_copy(row_hbm.at[pl.ds(cbase + nb * IB, IB)],
                         rowv.at[nbbuf], isem)
        pltpu.async_copy(col_hbm.at[pl.ds(cbase + nb * IB, IB)],
                         colv.at[nbbuf], isem)

        pltpu.async_copy(x_hbm.at[colv.at[bbuf].at[0]], rows.at[0], gsem)

        def _chunk(i, carry2):
            buf = lax.rem(i, 2)
            nbuf = lax.rem(i + 1, 2)
            nxt = jnp.minimum(i + 1, IB - 1)
            pltpu.make_async_copy(x_hbm.at[colv.at[bbuf].at[i]], rows.at[buf],
                                  gsem).wait()
            pltpu.async_copy(x_hbm.at[colv.at[bbuf].at[nxt]], rows.at[nbuf],
                             gsem)
            pltpu.sync_copy(rows.at[buf], acc.at[rowv.at[bbuf].at[i]],
                            add=True)
            return carry2
        lax.fori_loop(0, IB, _chunk, 0)
        pltpu.make_async_copy(x_hbm.at[colv.at[bbuf].at[0]],
                              rows.at[lax.rem(IB, 2)], gsem).wait()
        return carry
    lax.fori_loop(0, nblocks, _block, 0)
    pltpu.make_async_copy(row_hbm.at[pl.ds(cbase, IB)], rowv.at[0], isem).wait()
    pltpu.make_async_copy(col_hbm.at[pl.ds(cbase, IB)], colv.at[0], isem).wait()
    edge_scope.__exit__(None, None, None)

    with jax.named_scope("writeback"):
        plsc.subcore_barrier()
        iota16 = jax.lax.iota(jnp.int32, 16)
        obase = c * N_PAD + s * ROWS_PER_TILE
        nslab = ROWS_PER_TILE // CHUNK
        for k in range(nslab):
            kb = k % 2
            if k >= 2:
                pltpu.make_async_copy(rows.at[kb], out_hbm.at[widx.at[kb]],
                                      ssem).wait()
            pltpu.sync_copy(acc.at[pl.ds(s * ROWS_PER_TILE + k * CHUNK, CHUNK)],
                            rows.at[kb])
            for j in range(CHUNK // LANES):
                widx[kb, pl.ds(j * LANES, LANES)] = (
                    obase + k * CHUNK + j * LANES + iota16)
            pltpu.async_copy(rows.at[kb], out_hbm.at[widx.at[kb]], ssem)
        for k in range(nslab - 2, nslab):
            pltpu.make_async_copy(rows.at[k % 2], out_hbm.at[widx.at[k % 2]],
                                  ssem).wait()


def _tc_matmul_body(p_ref, w_ref, o_ref):
    seg = p_ref[:N_NODES, :] + p_ref[N_PAD:N_PAD + N_NODES, :]
    o_ref[...] = lax.dot_general(
        seg, w_ref[...], (((1,), (1,)), ((), ())),
        preferred_element_type=jnp.float32,
        precision=lax.Precision.HIGHEST)


def kernel(x, edge_index, edge_attr, W_fc, W_edge, W_att):
    npad = E_PAD - N_EDGES
    pad_rows = N_NODES + jnp.arange(npad, dtype=jnp.int32) % (N_PAD - N_NODES)
    row = jnp.concatenate(
        [edge_index[0].astype(jnp.int32),
         pad_rows]).reshape(TOTAL_CHUNKS, CHUNK)
    col = jnp.concatenate(
        [edge_index[1].astype(jnp.int32),
         jnp.zeros((npad,), jnp.int32)]).reshape(TOTAL_CHUNKS, CHUNK)
    partials = _sc_segment_sum(row, col, x)
    z = pl.pallas_call(
        _tc_matmul_body,
        out_shape=jax.ShapeDtypeStruct((N_NODES, CH), jnp.float32),
    )(partials, W_fc)
    return z


# speedup vs baseline: 3.0027x; 2.7373x over previous
import functools

import jax
import jax.numpy as jnp
from jax import lax
from jax.experimental import pallas as pl
from jax.experimental.pallas import tpu as pltpu
from jax.experimental.pallas import tpu_sc as plsc

N_NODES = 10000
N_EDGES = 320000
CH = 128

NC = 2
NS = 16
NW = NC * NS
CHUNK = 128
TOTAL_CHUNKS = 2560
NCH = TOTAL_CHUNKS // NW
IB = 16
NBLK = NCH // IB
E_PAD = TOTAL_CHUNKS * CHUNK
N_PAD = 10240
ROWS_PER_TILE = N_PAD // NS
LANES = 16


_mesh = plsc.VectorSubcoreMesh(core_axis_name="c", subcore_axis_name="s")


@functools.partial(
    pl.kernel,
    out_type=jax.ShapeDtypeStruct((NC, N_PAD, CH), jnp.float32),
    mesh=_mesh,
    scratch_types=[
        pltpu.VMEM((2, IB, CHUNK), jnp.int32),
        pltpu.VMEM((2, IB, CHUNK), jnp.int32),
        pltpu.VMEM((2, CHUNK, CH), jnp.float32),
        pltpu.VMEM_SHARED((N_PAD, CH), jnp.float32),
        pltpu.SemaphoreType.DMA,
        pltpu.SemaphoreType.DMA,
    ],
)
def _sc_segment_sum(row_hbm, col_hbm, x_hbm, out_hbm,
                    rowv, colv, rows, acc, isem, gsem):
    c = lax.axis_index("c")
    s = lax.axis_index("s")
    cbase = (c * NS + s) * NCH

    with jax.named_scope("zero_init"):
        def _zero_row(i, carry):
            zero = jnp.zeros((LANES,), jnp.float32)
            for j in range(CH // LANES):
                rows[0, i, pl.ds(j * LANES, LANES)] = zero
            return carry
        lax.fori_loop(0, CHUNK, _zero_row, 0)
        for k in range(ROWS_PER_TILE // CHUNK):
            pltpu.sync_copy(rows.at[0],
                            acc.at[pl.ds(s * ROWS_PER_TILE + k * CHUNK, CHUNK)])
        plsc.subcore_barrier()

    edge_scope = jax.named_scope("edge_loop")
    edge_scope.__enter__()
    pltpu.async_copy(row_hbm.at[pl.ds(cbase, IB)], rowv.at[0], isem)
    pltpu.async_copy(col_hbm.at[pl.ds(cbase, IB)], colv.at[0], isem)

    def _block(b, carry):
        bbuf = lax.rem(b, 2)
        nbbuf = lax.rem(b + 1, 2)
        nb = jnp.minimum(b + 1, NBLK - 1)
        pltpu.make_async_copy(row_hbm.at[pl.ds(cbase, IB)], rowv.at[bbuf],
                              isem).wait()
        pltpu.make_async_copy(col_hbm.at[pl.ds(cbase, IB)], colv.at[bbuf],
                              isem).wait()
        pltpu.async_copy(row_hbm.at[pl.ds(cbase + nb * IB, IB)],
                         rowv.at[nbbuf], isem)
        pltpu.async_copy(col_hbm.at[pl.ds(cbase + nb * IB, IB)],
                         colv.at[nbbuf], isem)

        pltpu.async_copy(x_hbm.at[colv.at[bbuf].at[0]], rows.at[0], gsem)

        def _chunk(i, carry2):
            buf = lax.rem(i, 2)
            nbuf = lax.rem(i + 1, 2)
            nxt = jnp.minimum(i + 1, IB - 1)
            pltpu.make_async_copy(x_hbm.at[colv.at[bbuf].at[i]], rows.at[buf],
                                  gsem).wait()
            pltpu.async_copy(x_hbm.at[colv.at[bbuf].at[nxt]], rows.at[nbuf],
                             gsem)
            pltpu.sync_copy(rows.at[buf], acc.at[rowv.at[bbuf].at[i]],
                            add=True)
            return carry2
        lax.fori_loop(0, IB, _chunk, 0)
        pltpu.make_async_copy(x_hbm.at[colv.at[bbuf].at[0]],
                              rows.at[lax.rem(IB, 2)], gsem).wait()
        return carry
    lax.fori_loop(0, NBLK, _block, 0)
    pltpu.make_async_copy(row_hbm.at[pl.ds(cbase, IB)], rowv.at[0], isem).wait()
    pltpu.make_async_copy(col_hbm.at[pl.ds(cbase, IB)], colv.at[0], isem).wait()
    edge_scope.__exit__(None, None, None)

    with jax.named_scope("writeback"):
        plsc.subcore_barrier()
        pltpu.sync_copy(acc.at[pl.ds(s * ROWS_PER_TILE, ROWS_PER_TILE)],
                        out_hbm.at[c, pl.ds(s * ROWS_PER_TILE, ROWS_PER_TILE)])


def _tc_matmul_body(p_ref, w_ref, o_ref):
    seg = p_ref[0, :N_NODES, :] + p_ref[1, :N_NODES, :]
    o_ref[...] = lax.dot_general(
        seg, w_ref[...], (((1,), (1,)), ((), ())),
        preferred_element_type=jnp.float32,
        precision=lax.Precision.HIGHEST)


def kernel(x, edge_index, edge_attr, W_fc, W_edge, W_att):
    npad = E_PAD - N_EDGES
    spread = jnp.arange(npad, dtype=jnp.int32)
    pad_rows = N_NODES + spread % (N_PAD - N_NODES)
    pad_cols = spread % N_NODES
    row = jnp.concatenate(
        [edge_index[0].astype(jnp.int32),
         pad_rows]).reshape(TOTAL_CHUNKS, CHUNK)
    col = jnp.concatenate(
        [edge_index[1].astype(jnp.int32),
         pad_cols]).reshape(TOTAL_CHUNKS, CHUNK)
    partials = _sc_segment_sum(row, col, x)
    z = pl.pallas_call(
        _tc_matmul_body,
        out_shape=jax.ShapeDtypeStruct((N_NODES, CH), jnp.float32),
    )(partials, W_fc)
    return z


# speedup vs baseline: 3.9156x; 1.3040x over previous
import functools

import jax
import jax.numpy as jnp
from jax import lax
from jax.experimental import pallas as pl
from jax.experimental.pallas import tpu as pltpu
from jax.experimental.pallas import tpu_sc as plsc

N_NODES = 10000
N_EDGES = 320000
CH = 128

NC = 2
NS = 16
NW = NC * NS
CHUNK = 80
TOTAL_CHUNKS = 4096
NCH = TOTAL_CHUNKS // NW
IB = 8
NBLK = NCH // IB
GDEPTH = 3
E_PAD = TOTAL_CHUNKS * CHUNK
N_PAD = 10240
ROWS_PER_TILE = N_PAD // NS
LANES = 16


_mesh = plsc.VectorSubcoreMesh(core_axis_name="c", subcore_axis_name="s")


@functools.partial(
    pl.kernel,
    out_type=jax.ShapeDtypeStruct((NC, N_PAD, CH), jnp.float32),
    mesh=_mesh,
    scratch_types=[
        pltpu.VMEM((3, IB, CHUNK), jnp.int32),
        pltpu.VMEM((3, IB, CHUNK), jnp.int32),
        pltpu.VMEM((4, CHUNK, CH), jnp.float32),
        pltpu.VMEM_SHARED((N_PAD, CH), jnp.float32),
        pltpu.SemaphoreType.DMA,
        pltpu.SemaphoreType.DMA,
    ],
)
def _sc_segment_sum(row_hbm, col_hbm, x_hbm, out_hbm,
                    rowv, colv, rows, acc, isem, gsem):
    c = lax.axis_index("c")
    s = lax.axis_index("s")
    cbase = (c * NS + s) * NCH

    with jax.named_scope("zero_init"):
        def _zero_row(i, carry):
            zero = jnp.zeros((LANES,), jnp.float32)
            for j in range(CH // LANES):
                rows[0, i, pl.ds(j * LANES, LANES)] = zero
            return carry
        lax.fori_loop(0, CHUNK, _zero_row, 0)
        for k in range(ROWS_PER_TILE // CHUNK):
            pltpu.sync_copy(rows.at[0],
                            acc.at[pl.ds(s * ROWS_PER_TILE + k * CHUNK, CHUNK)])
        plsc.subcore_barrier()

    edge_scope = jax.named_scope("edge_loop")
    edge_scope.__enter__()
    pltpu.async_copy(row_hbm.at[pl.ds(cbase, IB)], rowv.at[0], isem)
    pltpu.async_copy(col_hbm.at[pl.ds(cbase, IB)], colv.at[0], isem)
    pltpu.async_copy(row_hbm.at[pl.ds(cbase + IB, IB)], rowv.at[1], isem)
    pltpu.async_copy(col_hbm.at[pl.ds(cbase + IB, IB)], colv.at[1], isem)
    pltpu.make_async_copy(row_hbm.at[pl.ds(cbase, IB)], rowv.at[0], isem).wait()
    pltpu.make_async_copy(col_hbm.at[pl.ds(cbase, IB)], colv.at[0], isem).wait()
    for g in range(GDEPTH):
        pltpu.async_copy(x_hbm.at[colv.at[0].at[g]], rows.at[g], gsem)

    def _block(b, carry):
        b3 = lax.rem(b, 3)

        @pl.when(b + 1 <= NBLK - 1)
        def _wait_next_idx():
            pltpu.make_async_copy(row_hbm.at[pl.ds(cbase, IB)],
                                  rowv.at[lax.rem(b + 1, 3)], isem).wait()
            pltpu.make_async_copy(col_hbm.at[pl.ds(cbase, IB)],
                                  colv.at[lax.rem(b + 1, 3)], isem).wait()

        @pl.when(b + 2 <= NBLK - 1)
        def _prefetch_idx():
            nb = b + 2
            pltpu.async_copy(row_hbm.at[pl.ds(cbase + nb * IB, IB)],
                             rowv.at[lax.rem(nb, 3)], isem)
            pltpu.async_copy(col_hbm.at[pl.ds(cbase + nb * IB, IB)],
                             colv.at[lax.rem(nb, 3)], isem)

        def _chunk(i, carry2):
            g = b * IB + i
            pltpu.make_async_copy(x_hbm.at[colv.at[b3].at[i]],
                                  rows.at[lax.rem(g, 4)], gsem).wait()
            gg = g + GDEPTH

            @pl.when(gg <= NCH - 1)
            def _issue_gather():
                bb = lax.div(gg, IB)
                pltpu.async_copy(
                    x_hbm.at[colv.at[lax.rem(bb, 3)].at[lax.rem(gg, IB)]],
                    rows.at[lax.rem(gg, 4)], gsem)

            pltpu.sync_copy(rows.at[lax.rem(g, 4)], acc.at[rowv.at[b3].at[i]],
                            add=True)
            return carry2
        lax.fori_loop(0, IB, _chunk, 0)
        return carry
    lax.fori_loop(0, NBLK, _block, 0)
    edge_scope.__exit__(None, None, None)

    with jax.named_scope("writeback"):
        plsc.subcore_barrier()
        pltpu.sync_copy(acc.at[pl.ds(s * ROWS_PER_TILE, ROWS_PER_TILE)],
                        out_hbm.at[c, pl.ds(s * ROWS_PER_TILE, ROWS_PER_TILE)])


def _tc_matmul_body(p_ref, w_ref, o_ref):
    seg = p_ref[0, :N_NODES, :] + p_ref[1, :N_NODES, :]
    o_ref[...] = lax.dot_general(
        seg, w_ref[...], (((1,), (1,)), ((), ())),
        preferred_element_type=jnp.float32,
        precision=lax.Precision.HIGHEST)


def kernel(x, edge_index, edge_attr, W_fc, W_edge, W_att):
    npad = E_PAD - N_EDGES
    spread = jnp.arange(npad, dtype=jnp.int32)
    pad_rows = N_NODES + spread % (N_PAD - N_NODES)
    pad_cols = spread % N_NODES
    row = jnp.concatenate(
        [edge_index[0].astype(jnp.int32),
         pad_rows]).reshape(TOTAL_CHUNKS, CHUNK)
    col = jnp.concatenate(
        [edge_index[1].astype(jnp.int32),
         pad_cols]).reshape(TOTAL_CHUNKS, CHUNK)
    partials = _sc_segment_sum(row, col, x)
    z = pl.pallas_call(
        _tc_matmul_body,
        out_shape=jax.ShapeDtypeStruct((N_NODES, CH), jnp.float32),
    )(partials, W_fc)
    return z


# speedup vs baseline: 3.9288x; 1.0034x over previous
import functools

import jax
import jax.numpy as jnp
from jax import lax
from jax.experimental import pallas as pl
from jax.experimental.pallas import tpu as pltpu
from jax.experimental.pallas import tpu_sc as plsc

N_NODES = 10000
N_EDGES = 320000
CH = 128

NC = 2
NS = 16
NW = NC * NS
CHUNK = 80
TOTAL_CHUNKS = 4096
NCH = TOTAL_CHUNKS // NW
IB = 8
NBLK = NCH // IB
GDEPTH = 3
E_PAD = TOTAL_CHUNKS * CHUNK
N_PAD = 10240
ROWS_PER_TILE = N_PAD // NS
LANES = 16


_mesh = plsc.VectorSubcoreMesh(core_axis_name="c", subcore_axis_name="s")


@functools.partial(
    pl.kernel,
    out_type=jax.ShapeDtypeStruct((NC, N_PAD, CH), jnp.float32),
    mesh=_mesh,
    scratch_types=[
        pltpu.VMEM((3, IB, CHUNK), jnp.int32),
        pltpu.VMEM((3, IB, CHUNK), jnp.int32),
        pltpu.VMEM((4, CHUNK, CH), jnp.float32),
        pltpu.VMEM_SHARED((N_PAD, CH), jnp.float32),
        pltpu.SemaphoreType.DMA,
        pltpu.SemaphoreType.DMA,
    ],
)
def _sc_segment_sum(row_hbm, col_hbm, x_hbm, out_hbm,
                    rowv, colv, rows, acc, isem, gsem):
    c = lax.axis_index("c")
    s = lax.axis_index("s")
    cbase = (c * NS + s) * NCH

    with jax.named_scope("zero_init"):
        def _zero_row(i, carry):
            zero = jnp.zeros((LANES,), jnp.float32)
            for j in range(CH // LANES):
                rows[0, i, pl.ds(j * LANES, LANES)] = zero
            return carry
        lax.fori_loop(0, CHUNK, _zero_row, 0)
        for k in range(ROWS_PER_TILE // CHUNK):
            pltpu.sync_copy(rows.at[0],
                            acc.at[pl.ds(s * ROWS_PER_TILE + k * CHUNK, CHUNK)])
        plsc.subcore_barrier()

    edge_scope = jax.named_scope("edge_loop")
    edge_scope.__enter__()
    pltpu.async_copy(row_hbm.at[pl.ds(cbase, IB)], rowv.at[0], isem)
    pltpu.async_copy(col_hbm.at[pl.ds(cbase, IB)], colv.at[0], isem)
    pltpu.async_copy(row_hbm.at[pl.ds(cbase + IB, IB)], rowv.at[1], isem)
    pltpu.async_copy(col_hbm.at[pl.ds(cbase + IB, IB)], colv.at[1], isem)
    pltpu.make_async_copy(row_hbm.at[pl.ds(cbase, IB)], rowv.at[0], isem).wait()
    pltpu.make_async_copy(col_hbm.at[pl.ds(cbase, IB)], colv.at[0], isem).wait()
    for g in range(GDEPTH):
        pltpu.async_copy(x_hbm.at[colv.at[0].at[g]], rows.at[g], gsem)

    def _block(b, carry):
        b3 = lax.rem(b, 3)

        @pl.when(b + 1 <= NBLK - 1)
        def _wait_next_idx():
            pltpu.make_async_copy(row_hbm.at[pl.ds(cbase, IB)],
                                  rowv.at[lax.rem(b + 1, 3)], isem).wait()
            pltpu.make_async_copy(col_hbm.at[pl.ds(cbase, IB)],
                                  colv.at[lax.rem(b + 1, 3)], isem).wait()

        @pl.when(b + 2 <= NBLK - 1)
        def _prefetch_idx():
            nb = b + 2
            pltpu.async_copy(row_hbm.at[pl.ds(cbase + nb * IB, IB)],
                             rowv.at[lax.rem(nb, 3)], isem)
            pltpu.async_copy(col_hbm.at[pl.ds(cbase + nb * IB, IB)],
                             colv.at[lax.rem(nb, 3)], isem)

        def _chunk(i, carry2):
            g = b * IB + i
            pltpu.make_async_copy(x_hbm.at[colv.at[b3].at[i]],
                                  rows.at[lax.rem(g, 4)], gsem).wait()
            gg = g + GDEPTH

            @pl.when(gg <= NCH - 1)
            def _issue_gather():
                bb = lax.div(gg, IB)
                pltpu.async_copy(
                    x_hbm.at[colv.at[lax.rem(bb, 3)].at[lax.rem(gg, IB)]],
                    rows.at[lax.rem(gg, 4)], gsem)

            pltpu.sync_copy(rows.at[lax.rem(g, 4)], acc.at[rowv.at[b3].at[i]],
                            add=True)
            return carry2
        lax.fori_loop(0, IB, _chunk, 0)
        return carry
    lax.fori_loop(0, NBLK, _block, 0)
    edge_scope.__exit__(None, None, None)

    with jax.named_scope("writeback"):
        plsc.subcore_barrier()
        pltpu.sync_copy(acc.at[pl.ds(s * ROWS_PER_TILE, ROWS_PER_TILE)],
                        out_hbm.at[c, pl.ds(s * ROWS_PER_TILE, ROWS_PER_TILE)])


TC_BLOCK = 2000


def _tc_matmul_body(p_ref, w_ref, o_ref):
    seg = p_ref[0] + p_ref[1]
    o_ref[...] = lax.dot_general(
        seg, w_ref[...], (((1,), (1,)), ((), ())),
        preferred_element_type=jnp.float32,
        precision=lax.Precision.HIGHEST)


def kernel(x, edge_index, edge_attr, W_fc, W_edge, W_att):
    npad = E_PAD - N_EDGES
    spread = jnp.arange(npad, dtype=jnp.int32)
    pad_rows = N_NODES + spread % (N_PAD - N_NODES)
    pad_cols = spread % N_NODES
    row = jnp.concatenate(
        [edge_index[0].astype(jnp.int32),
         pad_rows]).reshape(TOTAL_CHUNKS, CHUNK)
    col = jnp.concatenate(
        [edge_index[1].astype(jnp.int32),
         pad_cols]).reshape(TOTAL_CHUNKS, CHUNK)
    partials = _sc_segment_sum(row, col, x)
    z = pl.pallas_call(
        _tc_matmul_body,
        grid=(N_NODES // TC_BLOCK,),
        in_specs=[
            pl.BlockSpec((2, TC_BLOCK, CH), lambda i: (0, i, 0)),
            pl.BlockSpec((CH, CH), lambda i: (0, 0)),
        ],
        out_specs=pl.BlockSpec((TC_BLOCK, CH), lambda i: (i, 0)),
        out_shape=jax.ShapeDtypeStruct((N_NODES, CH), jnp.float32),
    )(partials, W_fc)
    return z


# speedup vs baseline: 4.0599x; 1.0334x over previous
import functools

import jax
import jax.numpy as jnp
from jax import lax
from jax.experimental import pallas as pl
from jax.experimental.pallas import tpu as pltpu
from jax.experimental.pallas import tpu_sc as plsc

N_NODES = 10000
N_EDGES = 320000
CH = 128

NC = 2
NS = 16
NW = NC * NS
CHUNK = 80
E_PER_TILE = N_EDGES // NW
NCH = E_PER_TILE // CHUNK
IB = 5
IBE = IB * CHUNK
NBLK = NCH // IB
GDEPTH = 3
N_PAD = 10240
ROWS_PER_TILE = N_PAD // NS
LANES = 16


_mesh = plsc.VectorSubcoreMesh(core_axis_name="c", subcore_axis_name="s")


@functools.partial(
    pl.kernel,
    out_type=jax.ShapeDtypeStruct((NC, N_PAD, CH), jnp.float32),
    mesh=_mesh,
    scratch_types=[
        pltpu.VMEM((3 * IBE,), jnp.int32),
        pltpu.VMEM((3 * IBE,), jnp.int32),
        pltpu.VMEM((IB, CHUNK), jnp.int32),
        pltpu.VMEM((4, CHUNK, CH), jnp.float32),
        pltpu.VMEM_SHARED((N_PAD, CH), jnp.float32),
        pltpu.SemaphoreType.DMA,
        pltpu.SemaphoreType.DMA,
    ],
)
def _sc_segment_sum(row_hbm, col_hbm, x_hbm, out_hbm,
                    rowv, colv, rowv2, rows, acc, isem, gsem):
    c = lax.axis_index("c")
    s = lax.axis_index("s")
    cbase = (c * NS + s) * E_PER_TILE

    with jax.named_scope("zero_init"):
        def _zero_row(i, carry):
            zero = jnp.zeros((LANES,), jnp.float32)
            for j in range(CH // LANES):
                rows[0, i, pl.ds(j * LANES, LANES)] = zero
            return carry
        lax.fori_loop(0, CHUNK, _zero_row, 0)
        for k in range(ROWS_PER_TILE // CHUNK):
            pltpu.sync_copy(rows.at[0],
                            acc.at[pl.ds(s * ROWS_PER_TILE + k * CHUNK, CHUNK)])
        plsc.subcore_barrier()

    edge_scope = jax.named_scope("edge_loop")
    edge_scope.__enter__()
    pltpu.async_copy(row_hbm.at[pl.ds(cbase, IBE)], rowv.at[pl.ds(0, IBE)],
                     isem)
    pltpu.async_copy(col_hbm.at[pl.ds(cbase, IBE)], colv.at[pl.ds(0, IBE)],
                     isem)
    pltpu.async_copy(row_hbm.at[pl.ds(cbase + IBE, IBE)],
                     rowv.at[pl.ds(IBE, IBE)], isem)
    pltpu.async_copy(col_hbm.at[pl.ds(cbase + IBE, IBE)],
                     colv.at[pl.ds(IBE, IBE)], isem)
    pltpu.make_async_copy(row_hbm.at[pl.ds(cbase, IBE)],
                          rowv.at[pl.ds(0, IBE)], isem).wait()
    pltpu.make_async_copy(col_hbm.at[pl.ds(cbase, IBE)],
                          colv.at[pl.ds(0, IBE)], isem).wait()
    for g in range(GDEPTH):
        pltpu.async_copy(x_hbm.at[colv.at[pl.ds(g * CHUNK, CHUNK)]],
                         rows.at[g], gsem)

    def _block(b, carry):
        boff = lax.rem(b, 3) * IBE

        @pl.when(b + 1 <= NBLK - 1)
        def _wait_next_idx():
            noff = lax.rem(b + 1, 3) * IBE
            pltpu.make_async_copy(row_hbm.at[pl.ds(cbase, IBE)],
                                  rowv.at[pl.ds(noff, IBE)], isem).wait()
            pltpu.make_async_copy(col_hbm.at[pl.ds(cbase, IBE)],
                                  colv.at[pl.ds(noff, IBE)], isem).wait()

        @pl.when(b + 2 <= NBLK - 1)
        def _prefetch_idx():
            nb = b + 2
            noff = lax.rem(nb, 3) * IBE
            pltpu.async_copy(row_hbm.at[pl.ds(cbase + nb * IBE, IBE)],
                             rowv.at[pl.ds(noff, IBE)], isem)
            pltpu.async_copy(col_hbm.at[pl.ds(cbase + nb * IBE, IBE)],
                             colv.at[pl.ds(noff, IBE)], isem)

        for j in range(IB):
            for k in range(CHUNK // LANES):
                rowv2[j, pl.ds(k * LANES, LANES)] = (
                    rowv[pl.ds(boff + j * CHUNK + k * LANES, LANES)])

        def _chunk(i, carry2):
            g = b * IB + i
            pltpu.make_async_copy(
                x_hbm.at[colv.at[pl.ds(boff + i * CHUNK, CHUNK)]],
                rows.at[lax.rem(g, 4)], gsem).wait()
            gg = g + GDEPTH

            @pl.when(gg <= NCH - 1)
            def _issue_gather():
                goff = (lax.rem(lax.div(gg, IB), 3) * IBE
                        + lax.rem(gg, IB) * CHUNK)
                pltpu.async_copy(
                    x_hbm.at[colv.at[pl.ds(goff, CHUNK)]],
                    rows.at[lax.rem(gg, 4)], gsem)

            pltpu.sync_copy(rows.at[lax.rem(g, 4)], acc.at[rowv2.at[i]],
                            add=True)
            return carry2
        lax.fori_loop(0, IB, _chunk, 0)
        return carry
    lax.fori_loop(0, NBLK, _block, 0)
    edge_scope.__exit__(None, None, None)

    with jax.named_scope("writeback"):
        plsc.subcore_barrier()
        pltpu.sync_copy(acc.at[pl.ds(s * ROWS_PER_TILE, ROWS_PER_TILE)],
                        out_hbm.at[c, pl.ds(s * ROWS_PER_TILE, ROWS_PER_TILE)])


TC_BLOCK = 2000


def _tc_matmul_body(p_ref, w_ref, o_ref):
    seg = p_ref[0] + p_ref[1]
    o_ref[...] = lax.dot_general(
        seg, w_ref[...], (((1,), (1,)), ((), ())),
        preferred_element_type=jnp.float32,
        precision=lax.Precision.HIGHEST)


def kernel(x, edge_index, edge_attr, W_fc, W_edge, W_att):
    row = edge_index[0].astype(jnp.int32)
    col = edge_index[1].astype(jnp.int32)
    partials = _sc_segment_sum(row, col, x)
    z = pl.pallas_call(
        _tc_matmul_body,
        grid=(N_NODES // TC_BLOCK,),
        in_specs=[
            pl.BlockSpec((2, TC_BLOCK, CH), lambda i: (0, i, 0)),
            pl.BlockSpec((CH, CH), lambda i: (0, 0)),
        ],
        out_specs=pl.BlockSpec((TC_BLOCK, CH), lambda i: (i, 0)),
        out_shape=jax.ShapeDtypeStruct((N_NODES, CH), jnp.float32),
    )(partials, W_fc)
    return z


# speedup vs baseline: 4.1200x; 1.0148x over previous
import functools

import jax
import jax.numpy as jnp
from jax import lax
from jax.experimental import pallas as pl
from jax.experimental.pallas import tpu as pltpu
from jax.experimental.pallas import tpu_sc as plsc

N_NODES = 10000
N_EDGES = 320000
CH = 128

NC = 2
NS = 16
NW = NC * NS
CHUNK = 80
E_PER_TILE = N_EDGES // NW
NCH = E_PER_TILE // CHUNK
IB = 5
IBE = IB * CHUNK
NBLK = NCH // IB
GDEPTH = 3
N_PAD = 10240
ROWS_PER_TILE = N_PAD // NS
LANES = 16


_mesh = plsc.VectorSubcoreMesh(core_axis_name="c", subcore_axis_name="s")


@functools.partial(
    pl.kernel,
    out_type=jax.ShapeDtypeStruct((NC, N_PAD, CH), jnp.float32),
    mesh=_mesh,
    scratch_types=[
        pltpu.VMEM((3 * IBE,), jnp.int32),
        pltpu.VMEM((3 * IBE,), jnp.int32),
        pltpu.VMEM((IB, CHUNK), jnp.int32),
        pltpu.VMEM((4, CHUNK, CH), jnp.float32),
        pltpu.VMEM_SHARED((N_PAD, CH), jnp.float32),
        pltpu.SemaphoreType.DMA,
        pltpu.SemaphoreType.DMA,
    ],
)
def _sc_segment_sum(row_hbm, col_hbm, x_hbm, out_hbm,
                    rowv, colv, rowv2, rows, acc, isem, gsem):
    c = lax.axis_index("c")
    s = lax.axis_index("s")
    cbase = (c * NS + s) * E_PER_TILE

    with jax.named_scope("zero_init"):
        def _zero_row(i, carry):
            zero = jnp.zeros((LANES,), jnp.float32)
            for j in range(CH // LANES):
                rows[0, i, pl.ds(j * LANES, LANES)] = zero
            return carry
        lax.fori_loop(0, CHUNK, _zero_row, 0)
        for k in range(ROWS_PER_TILE // CHUNK):
            pltpu.sync_copy(rows.at[0],
                            acc.at[pl.ds(s * ROWS_PER_TILE + k * CHUNK, CHUNK)])
        plsc.subcore_barrier()

    edge_scope = jax.named_scope("edge_loop")
    edge_scope.__enter__()
    pltpu.async_copy(row_hbm.at[pl.ds(cbase, IBE)], rowv.at[pl.ds(0, IBE)],
                     isem)
    pltpu.async_copy(col_hbm.at[pl.ds(cbase, IBE)], colv.at[pl.ds(0, IBE)],
                     isem)
    pltpu.async_copy(row_hbm.at[pl.ds(cbase + IBE, IBE)],
                     rowv.at[pl.ds(IBE, IBE)], isem)
    pltpu.async_copy(col_hbm.at[pl.ds(cbase + IBE, IBE)],
                     colv.at[pl.ds(IBE, IBE)], isem)
    pltpu.make_async_copy(row_hbm.at[pl.ds(cbase, IBE)],
                          rowv.at[pl.ds(0, IBE)], isem).wait()
    pltpu.make_async_copy(col_hbm.at[pl.ds(cbase, IBE)],
                          colv.at[pl.ds(0, IBE)], isem).wait()
    for g in range(GDEPTH):
        pltpu.async_copy(x_hbm.at[colv.at[pl.ds(g * CHUNK, CHUNK)]],
                         rows.at[g], gsem)

    def _block(b, carry):
        boff = lax.rem(b, 3) * IBE

        @pl.when(b + 1 <= NBLK - 1)
        def _wait_next_idx():
            noff = lax.rem(b + 1, 3) * IBE
            pltpu.make_async_copy(row_hbm.at[pl.ds(cbase, IBE)],
                                  rowv.at[pl.ds(noff, IBE)], isem).wait()
            pltpu.make_async_copy(col_hbm.at[pl.ds(cbase, IBE)],
                                  colv.at[pl.ds(noff, IBE)], isem).wait()

        @pl.when(b + 2 <= NBLK - 1)
        def _prefetch_idx():
            nb = b + 2
            noff = lax.rem(nb, 3) * IBE
            pltpu.async_copy(row_hbm.at[pl.ds(cbase + nb * IBE, IBE)],
                             rowv.at[pl.ds(noff, IBE)], isem)
            pltpu.async_copy(col_hbm.at[pl.ds(cbase + nb * IBE, IBE)],
                             colv.at[pl.ds(noff, IBE)], isem)

        for j in range(IB):
            for k in range(CHUNK // LANES):
                rowv2[j, pl.ds(k * LANES, LANES)] = (
                    rowv[pl.ds(boff + j * CHUNK + k * LANES, LANES)])

        def _chunk(i, carry2):
            g = b * IB + i
            pltpu.make_async_copy(
                x_hbm.at[colv.at[pl.ds(boff + i * CHUNK, CHUNK)]],
                rows.at[lax.rem(g, 4)], gsem).wait()
            gg = g + GDEPTH

            @pl.when(gg <= NCH - 1)
            def _issue_gather():
                goff = (lax.rem(lax.div(gg, IB), 3) * IBE
                        + lax.rem(gg, IB) * CHUNK)
                pltpu.async_copy(
                    x_hbm.at[colv.at[pl.ds(goff, CHUNK)]],
                    rows.at[lax.rem(gg, 4)], gsem)

            pltpu.sync_copy(rows.at[lax.rem(g, 4)], acc.at[rowv2.at[i]],
                            add=True)
            return carry2
        lax.fori_loop(0, IB, _chunk, 0)
        return carry
    lax.fori_loop(0, NBLK, _block, 0)
    edge_scope.__exit__(None, None, None)

    with jax.named_scope("writeback"):
        plsc.subcore_barrier()
        pltpu.sync_copy(acc.at[pl.ds(s * ROWS_PER_TILE, ROWS_PER_TILE)],
                        out_hbm.at[c, pl.ds(s * ROWS_PER_TILE, ROWS_PER_TILE)])


TC_BLOCK = 2000


def _tc_matmul_body(p_ref, w_ref, o_ref):
    seg = p_ref[0] + p_ref[1]
    o_ref[...] = lax.dot_general(
        seg, w_ref[...], (((1,), (1,)), ((), ())),
        preferred_element_type=jnp.float32)


def kernel(x, edge_index, edge_attr, W_fc, W_edge, W_att):
    row = edge_index[0].astype(jnp.int32)
    col = edge_index[1].astype(jnp.int32)
    partials = _sc_segment_sum(row, col, x)
    z = pl.pallas_call(
        _tc_matmul_body,
        grid=(N_NODES // TC_BLOCK,),
        in_specs=[
            pl.BlockSpec((2, TC_BLOCK, CH), lambda i: (0, i, 0)),
            pl.BlockSpec((CH, CH), lambda i: (0, 0)),
        ],
        out_specs=pl.BlockSpec((TC_BLOCK, CH), lambda i: (i, 0)),
        out_shape=jax.ShapeDtypeStruct((N_NODES, CH), jnp.float32),
    )(partials, W_fc)
    return z
